# baseline (speedup 1.0000x reference)
import functools

import jax
import jax.numpy as jnp
from jax import lax
from jax.experimental import pallas as pl
from jax.experimental.pallas import tpu as pltpu
from jax.experimental.pallas import tpu_sc as plsc

_TOPK = 8
_NC = 2
_NS = 16
_NW = _NC * _NS


def _score_body(x_ref, w_ref, b_ref, s_ref):
    st = jax.lax.dot_general(
        w_ref[...], x_ref[...], (((0,), (1,)), ((), ())),
        preferred_element_type=jnp.float32,
    )
    s_ref[...] = jax.nn.sigmoid(st + b_ref[...])


@functools.partial(jax.jit, static_argnames=("block",))
def _scores(x, W, b, block=1024):
    t, d = x.shape
    n_e = W.shape[1]
    st = pl.pallas_call(
        _score_body,
        grid=(t // block,),
        in_specs=[
            pl.BlockSpec((block, d), lambda i: (i, 0)),
            pl.BlockSpec((d, n_e), lambda i: (0, 0)),
            pl.BlockSpec((n_e, 1), lambda i: (0, 0)),
        ],
        out_specs=pl.BlockSpec((n_e, block), lambda i: (0, i)),
        out_shape=jax.ShapeDtypeStruct((n_e, t), jnp.float32),
        compiler_params=pltpu.CompilerParams(
            dimension_semantics=("arbitrary",),
        ),
    )(x, W, b.reshape(n_e, 1))
    return st.T


def _sc_topk(s):
    t, n_e = s.shape
    tpw = t // _NW
    mesh = plsc.VectorSubcoreMesh(core_axis_name="c", subcore_axis_name="s")

    @functools.partial(
        pl.kernel,
        mesh=mesh,
        out_type=[
            jax.ShapeDtypeStruct((t * 16,), jnp.float32),
            jax.ShapeDtypeStruct((t * 16,), jnp.int32),
        ],
        scratch_types=[
            pltpu.VMEM((tpw * n_e,), jnp.float32),
            pltpu.VMEM((tpw * 16,), jnp.float32),
            pltpu.VMEM((tpw * 16,), jnp.int32),
        ],
        compiler_params=pltpu.CompilerParams(needs_layout_passes=False),
    )
    def k(s_hbm, g_hbm, i_hbm, s_v, g_v, i_v):
        wid = lax.axis_index("s") * _NC + lax.axis_index("c")
        base = wid * tpw
        pltpu.sync_copy(s_hbm.at[pl.ds(base * n_e, tpw * n_e)], s_v)

        def merge(ak, av, bk, bv):
            rbk = lax.rev(bk, (0,))
            rbv = lax.rev(bv, (0,))
            take = (ak > rbk) | ((ak == rbk) & (av < rbv))
            mk = jnp.where(take, ak, rbk)
            mv = jnp.where(take, av, rbv)
            return plsc.sort_key_val(mk, mv, descending=True)

        lane = lax.iota(jnp.int32, 16)

        def body(tk, carry):
            ks, vs = [], []
            for c in range(4):
                key = s_v[pl.ds(tk * n_e + c * 16, 16)]
                sk, sv = plsc.sort_key_val(key, lane + c * 16, descending=True)
                ks.append(sk)
                vs.append(sv)
            k01, v01 = merge(ks[0], vs[0], ks[1], vs[1])
            k23, v23 = merge(ks[2], vs[2], ks[3], vs[3])
            kf, vf = merge(k01, v01, k23, v23)
            top8 = jnp.where(lane < _TOPK, kf, 0.0)
            g_v[pl.ds(tk * 16, 16)] = top8 / jnp.sum(top8)
            i_v[pl.ds(tk * 16, 16)] = vf
            return carry

        lax.fori_loop(0, tpw, body, 0)
        pltpu.sync_copy(g_v, g_hbm.at[pl.ds(base * 16, tpw * 16)])
        pltpu.sync_copy(i_v, i_hbm.at[pl.ds(base * 16, tpw * 16)])

    g16, i16 = k(s.reshape(-1))
    return g16.reshape(t, 16)[:, :_TOPK], i16.reshape(t, 16)[:, :_TOPK]


def kernel(x, W, b):
    s = _scores(x, W, b)
    g_scores, indices = _sc_topk(s)
    return (g_scores, indices, s)


# speedup vs baseline: 1.9207x; 1.9207x over previous
import functools

import jax
import jax.numpy as jnp
from jax.experimental import pallas as pl
from jax.experimental.pallas import tpu as pltpu

_TOPK = 8


def _gate_body(x_ref, w_ref, b_ref, g_ref, i_ref, s_ref):
    st = jax.lax.dot_general(
        w_ref[...], x_ref[...], (((0,), (1,)), ((), ())),
        preferred_element_type=jnp.float32,
    )
    st = jax.nn.sigmoid(st + b_ref[...])
    s_ref[...] = st

    n_e = st.shape[0]
    expert = jax.lax.broadcasted_iota(jnp.int32, st.shape, 0)
    cur = st
    vals = []
    idxs = []
    for _ in range(_TOPK):
        m = jnp.max(cur, axis=0, keepdims=True)
        hit = cur == m
        idx = jnp.min(jnp.where(hit, expert, n_e), axis=0, keepdims=True)
        vals.append(m)
        idxs.append(idx)
        cur = jnp.where(expert == idx, -jnp.inf, cur)
    v = jnp.concatenate(vals, axis=0)
    g_ref[...] = v / jnp.sum(v, axis=0, keepdims=True)
    i_ref[...] = jnp.concatenate(idxs, axis=0)


@functools.partial(jax.jit, static_argnames=("block",))
def _gate(x, W, b, block=1024):
    t, d = x.shape
    n_e = W.shape[1]
    grid = (t // block,)
    gt, it, st = pl.pallas_call(
        _gate_body,
        grid=grid,
        in_specs=[
            pl.BlockSpec((block, d), lambda i: (i, 0)),
            pl.BlockSpec((d, n_e), lambda i: (0, 0)),
            pl.BlockSpec((n_e, 1), lambda i: (0, 0)),
        ],
        out_specs=[
            pl.BlockSpec((_TOPK, block), lambda i: (0, i)),
            pl.BlockSpec((_TOPK, block), lambda i: (0, i)),
            pl.BlockSpec((n_e, block), lambda i: (0, i)),
        ],
        out_shape=[
            jax.ShapeDtypeStruct((_TOPK, t), jnp.float32),
            jax.ShapeDtypeStruct((_TOPK, t), jnp.int32),
            jax.ShapeDtypeStruct((n_e, t), jnp.float32),
        ],
        compiler_params=pltpu.CompilerParams(
            dimension_semantics=("arbitrary",),
        ),
    )(x, W, b.reshape(n_e, 1))
    return gt.T, it.T, st.T


def kernel(x, W, b):
    g_scores, indices, s = _gate(x, W, b)
    return (g_scores, indices, s)


# speedup vs baseline: 1.9256x; 1.0026x over previous
import functools

import jax
import jax.numpy as jnp
from jax.experimental import pallas as pl
from jax.experimental.pallas import tpu as pltpu

_TOPK = 8


def _gate_body(x_ref, w_ref, b_ref, g_ref, i_ref, s_ref):
    st = jax.lax.dot_general(
        w_ref[...], x_ref[...], (((0,), (1,)), ((), ())),
        preferred_element_type=jnp.float32,
    )
    st = jax.nn.sigmoid(st + b_ref[...])
    s_ref[...] = st

    n_e = st.shape[0]
    expert = jax.lax.broadcasted_iota(jnp.int32, st.shape, 0)
    cur = st
    vals = []
    idxs = []
    for _ in range(_TOPK):
        m = jnp.max(cur, axis=0, keepdims=True)
        hit = cur == m
        idx = jnp.min(jnp.where(hit, expert, n_e), axis=0, keepdims=True)
        vals.append(m)
        idxs.append(idx)
        cur = jnp.where(expert == idx, -jnp.inf, cur)
    v = jnp.concatenate(vals, axis=0)
    g_ref[...] = v / jnp.sum(v, axis=0, keepdims=True)
    i_ref[...] = jnp.concatenate(idxs, axis=0)


@functools.partial(jax.jit, static_argnames=("block",))
def _gate(x, W, b, block=1024):
    t, d = x.shape
    n_e = W.shape[1]
    grid = (t // block,)
    gt, it, st = pl.pallas_call(
        _gate_body,
        grid=grid,
        in_specs=[
            pl.BlockSpec((block, d), lambda i: (i, 0)),
            pl.BlockSpec((d, n_e), lambda i: (0, 0)),
            pl.BlockSpec((n_e, 1), lambda i: (0, 0)),
        ],
        out_specs=[
            pl.BlockSpec((_TOPK, block), lambda i: (0, i)),
            pl.BlockSpec((_TOPK, block), lambda i: (0, i)),
            pl.BlockSpec((n_e, block), lambda i: (0, i)),
        ],
        out_shape=[
            jax.ShapeDtypeStruct((_TOPK, t), jnp.float32),
            jax.ShapeDtypeStruct((_TOPK, t), jnp.int32),
            jax.ShapeDtypeStruct((n_e, t), jnp.float32),
        ],
        compiler_params=pltpu.CompilerParams(
            dimension_semantics=("parallel",),
        ),
    )(x, W, b.reshape(n_e, 1))
    return gt.T, it.T, st.T


def kernel(x, W, b):
    g_scores, indices, s = _gate(x, W, b)
    return (g_scores, indices, s)
